# baseline (speedup 1.0000x reference)
import numpy as np

import jax
import jax.numpy as jnp
from jax import lax
from jax.experimental import pallas as pl
from jax.sharding import Mesh, PartitionSpec as P

_B, _NC, _IN = 32, 128, 12
_EH, _EO, _OUT = 32, 30, 12
_GC = 8
_NCH = _NC // _GC


def _prep_kernel(df_ref, gf_ref, d_ref, g_ref, x_ref, w1_ref, b1_ref,
                 b2_ref, b3_ref, adjf_ref, dn_ref, gn_ref, s_ref, tw_ref,
                 kb_ref, b2b_ref, b3b_ref):
    n = _B * _NC
    bl = d_ref.shape[0]
    bf = jnp.bfloat16
    df = df_ref[...].reshape(n, _NC)
    gf = gf_ref[...].reshape(n, _NC)
    md = jnp.mean(df, axis=0)
    vd = jnp.sum((df - md[None, :]) ** 2, axis=0) / (n - 1)
    rd = jax.lax.rsqrt(vd)
    mg = jnp.mean(gf, axis=0)
    vg = jnp.sum((gf - mg[None, :]) ** 2, axis=0) / (n - 1)
    rg = jax.lax.rsqrt(vg)

    d = d_ref[...].reshape(bl * _NC, _NC)
    g = g_ref[...].reshape(bl * _NC, _NC)
    adjf_ref[...] = (d != 0.0).astype(bf).reshape(bl, _NC, _NC)
    dn_ref[...] = ((d * rd[None, :] - (md * rd)[None, :])
                   .astype(bf).reshape(bl, _NC, _NC))
    gn_ref[...] = ((g * rg[None, :] - (mg * rg)[None, :])
                   .astype(bf).reshape(bl, _NC, _NC))

    w1 = w1_ref[...]
    at = w1[:, 0:12].T
    bt = w1[:, 13:25].T
    u = w1[:, 26:27]
    v = w1[:, 28:29]
    w = (w1[:, 12:13] + w1[:, 25:26] + w1[:, 27:28] + w1[:, 29:30]
         + w1[:, 30:31] + w1[:, 31:32])
    b1c = b1_ref[...].T
    kb_ref[...] = jnp.concatenate(
        [jnp.broadcast_to(u, (_EH, _NC)),
         jnp.broadcast_to(v, (_EH, _NC)),
         jnp.broadcast_to(b1c, (_EH, _NC))], axis=0).astype(bf)

    x2 = x_ref[...].reshape(bl * _NC, _IN)
    s_ref[...] = jnp.dot(x2, at).astype(bf).reshape(bl, _NC, _EH)
    t3 = jnp.dot(x2, bt).reshape(bl, _NC, _EH)
    wb = jnp.broadcast_to(w, (_EH, _NC))
    tw_ref[...] = (jnp.transpose(t3, (0, 2, 1)) + wb[None, :, :]).astype(bf)

    b2m = jnp.broadcast_to(b2_ref[...].T, (_EO, _NC))
    b2b_ref[...] = jnp.broadcast_to(
        b2m[:, None, :], (_EO, _GC, _NC)).reshape(_EO * _GC, _NC)
    b3b_ref[...] = jnp.broadcast_to(b3_ref[...].T, (_OUT, _NC))


def _main_kernel(kb_ref, bd_ref, b2b_ref, w3_ref, b3b_ref, adjf_ref, dn_ref,
                 gn_ref, s_ref, tw_ref, out_ref):
    adjf = adjf_ref[0]
    dn = dn_ref[0]
    gn = gn_ref[0]
    sb = s_ref[0]
    tw = tw_ref[0]
    kb = kb_ref[...]
    ub = kb[0:_EH]
    vb = kb[_EH:2 * _EH]
    b1b = kb[2 * _EH:3 * _EH]

    pre = (adjf[:, None, :] * (sb[:, :, None] + tw[None, :, :])
           + dn[:, None, :] * ub[None, :, :]
           + gn[:, None, :] * vb[None, :, :]
           + b1b[None, :, :])
    h = jax.nn.sigmoid(pre)

    bd = bd_ref[...]
    b2b = b2b_ref[...]
    add_acc = jnp.zeros((_EO * _GC, _NC), jnp.float32)
    subs = []
    for c in range(_NCH):
        hc = h[c * _GC:(c + 1) * _GC].reshape(_GC * _EH, _NC)
        ec = jax.nn.sigmoid(
            jnp.dot(bd, hc, preferred_element_type=jnp.float32) + b2b)
        add_acc = add_acc + ec
        subs.append(jnp.sum(ec.reshape(_EO, _GC, _NC), axis=2))
    add = jnp.sum(add_acc.reshape(_EO, _GC, _NC), axis=1)
    sub = jnp.concatenate(subs, axis=1)
    cmat = add - sub

    o = jax.nn.sigmoid(jnp.dot(w3_ref[...], cmat) + b3b_ref[...])
    out_ref[0] = o.T


def _device_body(bl, d_full, g_full, x_loc, w1, b1r, b2r, b3r, w3, bd):
    f32 = jnp.float32
    bf = jnp.bfloat16
    did = lax.axis_index("d")
    d_loc = lax.dynamic_slice_in_dim(d_full, did * bl, bl, 0)
    g_loc = lax.dynamic_slice_in_dim(g_full, did * bl, bl, 0)

    prep = pl.pallas_call(
        _prep_kernel,
        out_shape=(
            jax.ShapeDtypeStruct((bl, _NC, _NC), bf),
            jax.ShapeDtypeStruct((bl, _NC, _NC), bf),
            jax.ShapeDtypeStruct((bl, _NC, _NC), bf),
            jax.ShapeDtypeStruct((bl, _NC, _EH), bf),
            jax.ShapeDtypeStruct((bl, _EH, _NC), bf),
            jax.ShapeDtypeStruct((3 * _EH, _NC), bf),
            jax.ShapeDtypeStruct((_EO * _GC, _NC), f32),
            jax.ShapeDtypeStruct((_OUT, _NC), f32),
        ),
    )(d_full, g_full, d_loc, g_loc, x_loc, w1, b1r, b2r, b3r)
    adjf, dn, gn, s_all, tw_all, kb, b2b, b3b = prep

    const2 = lambda shape: pl.BlockSpec(shape, lambda b: (0, 0))
    out = pl.pallas_call(
        _main_kernel,
        grid=(bl,),
        in_specs=[
            const2((3 * _EH, _NC)),
            const2((_EO * _GC, _GC * _EH)),
            const2((_EO * _GC, _NC)),
            const2((_OUT, _EO)),
            const2((_OUT, _NC)),
            pl.BlockSpec((1, _NC, _NC), lambda b: (b, 0, 0)),
            pl.BlockSpec((1, _NC, _NC), lambda b: (b, 0, 0)),
            pl.BlockSpec((1, _NC, _NC), lambda b: (b, 0, 0)),
            pl.BlockSpec((1, _NC, _EH), lambda b: (b, 0, 0)),
            pl.BlockSpec((1, _EH, _NC), lambda b: (b, 0, 0)),
        ],
        out_specs=pl.BlockSpec((1, _NC, _OUT), lambda b: (b, 0, 0)),
        out_shape=jax.ShapeDtypeStruct((bl, _NC, _OUT), f32),
    )(kb, bd, b2b, w3, b3b, adjf, dn, gn, s_all, tw_all)
    return out


def kernel(x, adj_dist, adj_direct, wind_mean, wind_std, W1, b1, W2, b2, W3,
           b3):
    del wind_mean, wind_std
    f32 = jnp.float32
    bf = jnp.bfloat16

    eye = jnp.eye(_GC, dtype=f32)
    bd = jnp.einsum("gh,ok->oghk", eye, W2).reshape(
        _EO * _GC, _GC * _EH).astype(bf)

    devs = jax.devices()
    nd = 2 if len(devs) >= 2 and _B % 2 == 0 else 1
    bl = _B // nd
    mesh = Mesh(np.array(devs[:nd]), ("d",))

    body = lambda d_f, g_f, x_l, w1, b1r, b2r, b3r, w3, bdv: _device_body(
        bl, d_f, g_f, x_l, w1, b1r, b2r, b3r, w3, bdv)
    out = jax.shard_map(
        body,
        mesh=mesh,
        in_specs=(P(), P(), P("d"), P(), P(), P(), P(), P(), P()),
        out_specs=P("d"),
        check_vma=False,
    )(adj_dist, adj_direct, x, W1, b1.reshape(1, _EH), b2.reshape(1, _EO),
      b3.reshape(1, _OUT), W3, bd)
    return out


# speedup vs baseline: 6.7802x; 6.7802x over previous
import jax
import jax.numpy as jnp
from jax.experimental import pallas as pl

_B, _NC, _IN = 32, 128, 12
_EH, _EO, _OUT = 32, 30, 12
_GC = 8
_NCH = _NC // _GC
_BB = 2


def _prep_kernel(d_ref, g_ref, x_ref, w1_ref, b1_ref, b2_ref, b3_ref,
                 adjf_ref, dn_ref, gn_ref, s_ref, tw_ref, kb_ref, b2b_ref,
                 b3b_ref):
    n = _B * _NC
    bf = jnp.bfloat16
    d = d_ref[...].reshape(n, _NC)
    g = g_ref[...].reshape(n, _NC)
    md = jnp.mean(d, axis=0)
    vd = jnp.sum((d - md[None, :]) ** 2, axis=0) / (n - 1)
    rd = jax.lax.rsqrt(vd)
    mg = jnp.mean(g, axis=0)
    vg = jnp.sum((g - mg[None, :]) ** 2, axis=0) / (n - 1)
    rg = jax.lax.rsqrt(vg)

    adjf_ref[...] = (d != 0.0).astype(bf).reshape(_B, _NC, _NC)
    dn_ref[...] = ((d * rd[None, :] - (md * rd)[None, :])
                   .astype(bf).reshape(_B, _NC, _NC))
    gn_ref[...] = ((g * rg[None, :] - (mg * rg)[None, :])
                   .astype(bf).reshape(_B, _NC, _NC))

    w1 = w1_ref[...]
    at = w1[:, 0:12].T
    bt = w1[:, 13:25].T
    u = w1[:, 26:27]
    v = w1[:, 28:29]
    w = (w1[:, 12:13] + w1[:, 25:26] + w1[:, 27:28] + w1[:, 29:30]
         + w1[:, 30:31] + w1[:, 31:32])
    b1c = b1_ref[...].T
    kb_ref[...] = jnp.concatenate(
        [jnp.broadcast_to(u, (_EH, _NC)),
         jnp.broadcast_to(v, (_EH, _NC)),
         jnp.broadcast_to(b1c, (_EH, _NC))], axis=0).astype(bf)

    x2 = x_ref[...].reshape(n, _IN)
    s_ref[...] = jnp.dot(x2, at).astype(bf).reshape(_B, _NC, _EH)
    t3 = jnp.dot(x2, bt).reshape(_B, _NC, _EH)
    wb = jnp.broadcast_to(w, (_EH, _NC))
    tw_ref[...] = (jnp.transpose(t3, (0, 2, 1)) + wb[None, :, :]).astype(bf)

    b2m = jnp.broadcast_to(b2_ref[...].T, (_EO, _NC))
    b2b_ref[...] = jnp.broadcast_to(
        b2m[:, None, :], (_EO, _GC, _NC)).reshape(_EO * _GC, _NC)
    b3b_ref[...] = jnp.broadcast_to(b3_ref[...].T, (_OUT, _NC))


def _main_kernel(kb_ref, bd_ref, b2b_ref, w3_ref, b3b_ref, adjf_ref, dn_ref,
                 gn_ref, s_ref, tw_ref, out_ref):
    kb = kb_ref[...]
    ub = kb[0:_EH]
    vb = kb[_EH:2 * _EH]
    b1b = kb[2 * _EH:3 * _EH]
    bd = bd_ref[...]
    b2b = b2b_ref[...]

    for p in range(_BB):
        adjf = adjf_ref[p]
        dn = dn_ref[p]
        gn = gn_ref[p]
        sb = s_ref[p]
        tw = tw_ref[p]

        pre = (adjf[:, None, :] * (sb[:, :, None] + tw[None, :, :])
               + dn[:, None, :] * ub[None, :, :]
               + gn[:, None, :] * vb[None, :, :]
               + b1b[None, :, :])
        h = jax.nn.sigmoid(pre)

        add_acc = jnp.zeros((_EO * _GC, _NC), jnp.float32)
        subs = []
        for c in range(_NCH):
            hc = h[c * _GC:(c + 1) * _GC].reshape(_GC * _EH, _NC)
            ec = jax.nn.sigmoid(
                jnp.dot(bd, hc, preferred_element_type=jnp.float32) + b2b)
            add_acc = add_acc + ec
            subs.append(jnp.sum(ec.reshape(_EO, _GC, _NC), axis=2))
        add = jnp.sum(add_acc.reshape(_EO, _GC, _NC), axis=1)
        sub = jnp.concatenate(subs, axis=1)
        cmat = add - sub

        o = jax.nn.sigmoid(
            jnp.dot(w3_ref[...], cmat) + b3b_ref[...])
        out_ref[p] = o.T


def kernel(x, adj_dist, adj_direct, wind_mean, wind_std, W1, b1, W2, b2, W3,
           b3):
    del wind_mean, wind_std
    f32 = jnp.float32
    bf = jnp.bfloat16

    eye = jnp.eye(_GC, dtype=f32)
    bd = jnp.einsum("gh,ok->oghk", eye, W2).reshape(
        _EO * _GC, _GC * _EH).astype(bf)

    prep = pl.pallas_call(
        _prep_kernel,
        out_shape=(
            jax.ShapeDtypeStruct((_B, _NC, _NC), bf),
            jax.ShapeDtypeStruct((_B, _NC, _NC), bf),
            jax.ShapeDtypeStruct((_B, _NC, _NC), bf),
            jax.ShapeDtypeStruct((_B, _NC, _EH), bf),
            jax.ShapeDtypeStruct((_B, _EH, _NC), bf),
            jax.ShapeDtypeStruct((3 * _EH, _NC), bf),
            jax.ShapeDtypeStruct((_EO * _GC, _NC), f32),
            jax.ShapeDtypeStruct((_OUT, _NC), f32),
        ),
    )(adj_dist, adj_direct, x, W1, b1.reshape(1, _EH), b2.reshape(1, _EO),
      b3.reshape(1, _OUT))
    adjf, dn, gn, s_all, tw_all, kb, b2b, b3b = prep

    const2 = lambda shape: pl.BlockSpec(shape, lambda b: (0, 0))
    out = pl.pallas_call(
        _main_kernel,
        grid=(_B // _BB,),
        in_specs=[
            const2((3 * _EH, _NC)),
            const2((_EO * _GC, _GC * _EH)),
            const2((_EO * _GC, _NC)),
            const2((_OUT, _EO)),
            const2((_OUT, _NC)),
            pl.BlockSpec((_BB, _NC, _NC), lambda b: (b, 0, 0)),
            pl.BlockSpec((_BB, _NC, _NC), lambda b: (b, 0, 0)),
            pl.BlockSpec((_BB, _NC, _NC), lambda b: (b, 0, 0)),
            pl.BlockSpec((_BB, _NC, _EH), lambda b: (b, 0, 0)),
            pl.BlockSpec((_BB, _EH, _NC), lambda b: (b, 0, 0)),
        ],
        out_specs=pl.BlockSpec((_BB, _NC, _OUT), lambda b: (b, 0, 0)),
        out_shape=jax.ShapeDtypeStruct((_B, _NC, _OUT), f32),
    )(kb, bd, b2b, W3, b3b, adjf, dn, gn, s_all, tw_all)
    return out


# speedup vs baseline: 6.9664x; 1.0275x over previous
import jax
import jax.numpy as jnp
from jax.experimental import pallas as pl
from jax.experimental.pallas import tpu as pltpu

_B, _NC, _IN = 32, 128, 12
_EH, _EO, _OUT = 32, 30, 12
_GC = 8
_NCH = _NC // _GC
_BB = 2


def _fused_kernel(d_ref, g_ref, x_ref, w1_ref, b1_ref, b2_ref, b3_ref,
                  bd_ref, w3_ref, out_ref,
                  adjf_s, dn_s, gn_s, s_s, tw_s, kb_s, b2b_s, b3b_s):
    step = pl.program_id(0)
    n = _B * _NC
    bf = jnp.bfloat16

    @pl.when(step == 0)
    def _prep():
        d = d_ref[...].reshape(n, _NC)
        g = g_ref[...].reshape(n, _NC)
        md = jnp.mean(d, axis=0)
        vd = jnp.sum((d - md[None, :]) ** 2, axis=0) / (n - 1)
        rd = jax.lax.rsqrt(vd)
        mg = jnp.mean(g, axis=0)
        vg = jnp.sum((g - mg[None, :]) ** 2, axis=0) / (n - 1)
        rg = jax.lax.rsqrt(vg)

        adjf_s[...] = (d != 0.0).astype(bf).reshape(_B, _NC, _NC)
        dn_s[...] = ((d * rd[None, :] - (md * rd)[None, :])
                     .astype(bf).reshape(_B, _NC, _NC))
        gn_s[...] = ((g * rg[None, :] - (mg * rg)[None, :])
                     .astype(bf).reshape(_B, _NC, _NC))

        w1 = w1_ref[...]
        at = w1[:, 0:12].T
        bt = w1[:, 13:25].T
        u = w1[:, 26:27]
        v = w1[:, 28:29]
        w = (w1[:, 12:13] + w1[:, 25:26] + w1[:, 27:28] + w1[:, 29:30]
             + w1[:, 30:31] + w1[:, 31:32])
        b1c = b1_ref[...].T
        kb_s[...] = jnp.concatenate(
            [jnp.broadcast_to(u, (_EH, _NC)),
             jnp.broadcast_to(v, (_EH, _NC)),
             jnp.broadcast_to(b1c, (_EH, _NC))], axis=0).astype(bf)

        x2 = x_ref[...].reshape(n, _IN)
        s_s[...] = jnp.dot(x2, at).astype(bf).reshape(_B, _NC, _EH)
        t3 = jnp.dot(x2, bt).reshape(_B, _NC, _EH)
        wb = jnp.broadcast_to(w, (_EH, _NC))
        tw_s[...] = (jnp.transpose(t3, (0, 2, 1))
                     + wb[None, :, :]).astype(bf)

        b2m = jnp.broadcast_to(b2_ref[...].T, (_EO, _NC))
        b2b_s[...] = jnp.broadcast_to(
            b2m[:, None, :], (_EO, _GC, _NC)).reshape(_EO * _GC, _NC)
        b3b_s[...] = jnp.broadcast_to(b3_ref[...].T, (_OUT, _NC))

    kb = kb_s[...]
    ub = kb[0:_EH]
    vb = kb[_EH:2 * _EH]
    b1b = kb[2 * _EH:3 * _EH]
    bd = bd_ref[...]
    b2b = b2b_s[...]

    for p in range(_BB):
        idx = step * _BB + p
        adjf = adjf_s[idx]
        dn = dn_s[idx]
        gn = gn_s[idx]
        sb = s_s[idx]
        tw = tw_s[idx]

        pre = (adjf[:, None, :] * (sb[:, :, None] + tw[None, :, :])
               + dn[:, None, :] * ub[None, :, :]
               + gn[:, None, :] * vb[None, :, :]
               + b1b[None, :, :])
        h = jax.nn.sigmoid(pre)

        add_acc = jnp.zeros((_EO * _GC, _NC), jnp.float32)
        subs = []
        for c in range(_NCH):
            hc = h[c * _GC:(c + 1) * _GC].reshape(_GC * _EH, _NC)
            ec = jax.nn.sigmoid(
                jnp.dot(bd, hc, preferred_element_type=jnp.float32) + b2b)
            add_acc = add_acc + ec
            subs.append(jnp.sum(ec.reshape(_EO, _GC, _NC), axis=2))
        add = jnp.sum(add_acc.reshape(_EO, _GC, _NC), axis=1)
        sub = jnp.concatenate(subs, axis=1)
        cmat = add - sub

        o = jax.nn.sigmoid(
            jnp.dot(w3_ref[...], cmat) + b3b_s[...])
        out_ref[p] = o.T


def kernel(x, adj_dist, adj_direct, wind_mean, wind_std, W1, b1, W2, b2, W3,
           b3):
    del wind_mean, wind_std
    f32 = jnp.float32
    bf = jnp.bfloat16

    eye = jnp.eye(_GC, dtype=f32)
    bd = jnp.einsum("gh,ok->oghk", eye, W2).reshape(
        _EO * _GC, _GC * _EH).astype(bf)

    whole = lambda shape: pl.BlockSpec(shape, lambda b: tuple(
        0 for _ in shape))
    out = pl.pallas_call(
        _fused_kernel,
        grid=(_B // _BB,),
        in_specs=[
            whole((_B, _NC, _NC)),
            whole((_B, _NC, _NC)),
            whole((_B, _NC, _IN)),
            whole((_EH, _EH)),
            whole((1, _EH)),
            whole((1, _EO)),
            whole((1, _OUT)),
            whole((_EO * _GC, _GC * _EH)),
            whole((_OUT, _EO)),
        ],
        out_specs=pl.BlockSpec((_BB, _NC, _OUT), lambda b: (b, 0, 0)),
        out_shape=jax.ShapeDtypeStruct((_B, _NC, _OUT), f32),
        scratch_shapes=[
            pltpu.VMEM((_B, _NC, _NC), bf),
            pltpu.VMEM((_B, _NC, _NC), bf),
            pltpu.VMEM((_B, _NC, _NC), bf),
            pltpu.VMEM((_B, _NC, _EH), bf),
            pltpu.VMEM((_B, _EH, _NC), bf),
            pltpu.VMEM((3 * _EH, _NC), bf),
            pltpu.VMEM((_EO * _GC, _NC), f32),
            pltpu.VMEM((_OUT, _NC), f32),
        ],
    )(adj_dist, adj_direct, x, W1, b1.reshape(1, _EH), b2.reshape(1, _EO),
      b3.reshape(1, _OUT), bd, W3)
    return out


# speedup vs baseline: 7.7623x; 1.1142x over previous
import jax
import jax.numpy as jnp
from jax.experimental import pallas as pl
from jax.experimental.pallas import tpu as pltpu

_B, _NC, _IN = 32, 128, 12
_EH, _EO, _OUT = 32, 30, 12
_GC = 8
_NCH = _NC // _GC
_BB = 4


def _fused_kernel(d_ref, g_ref, x_ref, w1_ref, b1_ref, b2_ref, b3_ref,
                  w2_ref, w3_ref, out_ref,
                  adjf_s, dn_s, gn_s, s_s, tw_s, kb_s, b2b_s, b3b_s, bd_s):
    step = pl.program_id(0)
    n = _B * _NC
    bf = jnp.bfloat16

    @pl.when(step == 0)
    def _prep():
        d = d_ref[...].reshape(n, _NC)
        g = g_ref[...].reshape(n, _NC)
        md = jnp.mean(d, axis=0)
        vd = jnp.sum((d - md[None, :]) ** 2, axis=0) / (n - 1)
        rd = jax.lax.rsqrt(vd)
        mg = jnp.mean(g, axis=0)
        vg = jnp.sum((g - mg[None, :]) ** 2, axis=0) / (n - 1)
        rg = jax.lax.rsqrt(vg)

        adjf_s[...] = (d != 0.0).astype(bf).reshape(_B, _NC, _NC)
        dn_s[...] = ((d * rd[None, :] - (md * rd)[None, :])
                     .astype(bf).reshape(_B, _NC, _NC))
        gn_s[...] = ((g * rg[None, :] - (mg * rg)[None, :])
                     .astype(bf).reshape(_B, _NC, _NC))

        w1 = w1_ref[...]
        at = w1[:, 0:12].T
        bt = w1[:, 13:25].T
        u = w1[:, 26:27]
        v = w1[:, 28:29]
        w = (w1[:, 12:13] + w1[:, 25:26] + w1[:, 27:28] + w1[:, 29:30]
             + w1[:, 30:31] + w1[:, 31:32])
        b1c = b1_ref[...].T
        kb_s[...] = jnp.concatenate(
            [jnp.broadcast_to(u, (_EH, _NC)),
             jnp.broadcast_to(v, (_EH, _NC)),
             jnp.broadcast_to(b1c, (_EH, _NC))], axis=0).astype(bf)

        x2 = x_ref[...].reshape(n, _IN)
        s_s[...] = jnp.dot(x2, at).astype(bf).reshape(_B, _NC, _EH)
        t3 = jnp.dot(x2, bt).reshape(_B, _NC, _EH)
        wb = jnp.broadcast_to(w, (_EH, _NC))
        tw_s[...] = (jnp.transpose(t3, (0, 2, 1))
                     + wb[None, :, :]).astype(bf)

        b2m = jnp.broadcast_to(b2_ref[...].T, (_EO, _NC))
        b2b_s[...] = jnp.broadcast_to(
            b2m[:, None, :], (_EO, _GC, _NC)).reshape(_EO * _GC, _NC)
        b3b_s[...] = jnp.broadcast_to(b3_ref[...].T, (_OUT, _NC))

        w2rep = jnp.broadcast_to(
            w2_ref[...][:, None, :], (_EO, _GC, _EH)).reshape(
                _EO * _GC, _EH)
        gidx = jax.lax.broadcasted_iota(
            jnp.int32, (_EO * _GC, _EH), 0) % _GC
        bd_s[...] = jnp.concatenate(
            [jnp.where(gidx == gp, w2rep, 0.0) for gp in range(_GC)],
            axis=1).astype(jnp.bfloat16)

    kb = kb_s[...]
    ub = kb[0:_EH]
    vb = kb[_EH:2 * _EH]
    b1b = kb[2 * _EH:3 * _EH]
    bd = bd_s[...]
    b2b = b2b_s[...]

    for p in range(_BB):
        idx = step * _BB + p
        adjf = adjf_s[idx]
        dn = dn_s[idx]
        gn = gn_s[idx]
        sb = s_s[idx]
        tw = tw_s[idx]

        pre = (adjf[:, None, :] * (sb[:, :, None] + tw[None, :, :])
               + dn[:, None, :] * ub[None, :, :]
               + gn[:, None, :] * vb[None, :, :]
               + b1b[None, :, :])
        h = jax.nn.sigmoid(pre)

        add_acc = jnp.zeros((_EO * _GC, _NC), jnp.float32)
        subs = []
        for c in range(_NCH):
            hc = h[c * _GC:(c + 1) * _GC].reshape(_GC * _EH, _NC)
            ec = jax.nn.sigmoid(
                jnp.dot(bd, hc, preferred_element_type=jnp.float32) + b2b)
            add_acc = add_acc + ec
            subs.append(jnp.sum(ec.reshape(_EO, _GC, _NC), axis=2))
        add = jnp.sum(add_acc.reshape(_EO, _GC, _NC), axis=1)
        sub = jnp.concatenate(subs, axis=1)
        cmat = add - sub

        o = jax.nn.sigmoid(
            jnp.dot(w3_ref[...], cmat) + b3b_s[...])
        out_ref[p] = o.T


def kernel(x, adj_dist, adj_direct, wind_mean, wind_std, W1, b1, W2, b2, W3,
           b3):
    del wind_mean, wind_std
    f32 = jnp.float32
    bf = jnp.bfloat16

    whole = lambda shape: pl.BlockSpec(shape, lambda b: tuple(
        0 for _ in shape))
    out = pl.pallas_call(
        _fused_kernel,
        grid=(_B // _BB,),
        in_specs=[
            whole((_B, _NC, _NC)),
            whole((_B, _NC, _NC)),
            whole((_B, _NC, _IN)),
            whole((_EH, _EH)),
            whole((1, _EH)),
            whole((1, _EO)),
            whole((1, _OUT)),
            whole((_EO, _EH)),
            whole((_OUT, _EO)),
        ],
        out_specs=pl.BlockSpec((_BB, _NC, _OUT), lambda b: (b, 0, 0)),
        out_shape=jax.ShapeDtypeStruct((_B, _NC, _OUT), f32),
        scratch_shapes=[
            pltpu.VMEM((_B, _NC, _NC), bf),
            pltpu.VMEM((_B, _NC, _NC), bf),
            pltpu.VMEM((_B, _NC, _NC), bf),
            pltpu.VMEM((_B, _NC, _EH), bf),
            pltpu.VMEM((_B, _EH, _NC), bf),
            pltpu.VMEM((3 * _EH, _NC), bf),
            pltpu.VMEM((_EO * _GC, _NC), f32),
            pltpu.VMEM((_OUT, _NC), f32),
            pltpu.VMEM((_EO * _GC, _GC * _EH), bf),
        ],
    )(adj_dist, adj_direct, x, W1, b1.reshape(1, _EH), b2.reshape(1, _EO),
      b3.reshape(1, _OUT), W2, W3)
    return out


# speedup vs baseline: 7.9643x; 1.0260x over previous
import jax
import jax.numpy as jnp
from jax.experimental import pallas as pl
from jax.experimental.pallas import tpu as pltpu

_B, _NC, _IN = 32, 128, 12
_EH, _EO, _OUT = 32, 30, 12
_GC = 8
_NCH = _NC // _GC
_BB = 8


def _fused_kernel(d_ref, g_ref, x_ref, w1_ref, b1_ref, b2_ref, b3_ref,
                  w2_ref, w3_ref, out_ref,
                  adjf_s, dn_s, gn_s, s_s, tw_s, kb_s, b2b_s, b3b_s, bd_s):
    step = pl.program_id(0)
    n = _B * _NC
    bf = jnp.bfloat16

    @pl.when(step == 0)
    def _prep():
        d = d_ref[...].reshape(n, _NC)
        g = g_ref[...].reshape(n, _NC)
        md = jnp.mean(d, axis=0)
        vd = jnp.sum((d - md[None, :]) ** 2, axis=0) / (n - 1)
        rd = jax.lax.rsqrt(vd)
        mg = jnp.mean(g, axis=0)
        vg = jnp.sum((g - mg[None, :]) ** 2, axis=0) / (n - 1)
        rg = jax.lax.rsqrt(vg)

        adjf_s[...] = (d != 0.0).astype(bf).reshape(_B, _NC, _NC)
        dn_s[...] = ((d * rd[None, :] - (md * rd)[None, :])
                     .astype(bf).reshape(_B, _NC, _NC))
        gn_s[...] = ((g * rg[None, :] - (mg * rg)[None, :])
                     .astype(bf).reshape(_B, _NC, _NC))

        w1 = w1_ref[...]
        at = w1[:, 0:12].T
        bt = w1[:, 13:25].T
        u = w1[:, 26:27]
        v = w1[:, 28:29]
        w = (w1[:, 12:13] + w1[:, 25:26] + w1[:, 27:28] + w1[:, 29:30]
             + w1[:, 30:31] + w1[:, 31:32])
        b1c = b1_ref[...].T
        kb_s[...] = jnp.concatenate(
            [jnp.broadcast_to(u, (_EH, _NC)),
             jnp.broadcast_to(v, (_EH, _NC)),
             jnp.broadcast_to(b1c, (_EH, _NC))], axis=0).astype(bf)

        x2 = x_ref[...].reshape(n, _IN)
        s_s[...] = jnp.dot(x2, at).astype(bf).reshape(_B, _NC, _EH)
        t3 = jnp.dot(x2, bt).reshape(_B, _NC, _EH)
        wb = jnp.broadcast_to(w, (_EH, _NC))
        tw_s[...] = (jnp.transpose(t3, (0, 2, 1))
                     + wb[None, :, :]).astype(bf)

        b2m = jnp.broadcast_to(b2_ref[...].T, (_EO, _NC))
        b2b_s[...] = jnp.broadcast_to(
            b2m[:, None, :], (_EO, _GC, _NC)).reshape(_EO * _GC, _NC)
        b3b_s[...] = jnp.broadcast_to(b3_ref[...].T, (_OUT, _NC))

        w2rep = jnp.broadcast_to(
            w2_ref[...][:, None, :], (_EO, _GC, _EH)).reshape(
                _EO * _GC, _EH)
        gidx = jax.lax.broadcasted_iota(
            jnp.int32, (_EO * _GC, _EH), 0) % _GC
        bd_s[...] = jnp.concatenate(
            [jnp.where(gidx == gp, w2rep, 0.0) for gp in range(_GC)],
            axis=1).astype(jnp.bfloat16)

    kb = kb_s[...]
    ub = kb[0:_EH]
    vb = kb[_EH:2 * _EH]
    b1b = kb[2 * _EH:3 * _EH]
    bd = bd_s[...]
    b2b = b2b_s[...]

    outs = []
    for p in range(_BB):
        idx = step * _BB + p
        adjf = adjf_s[idx]
        dn = dn_s[idx]
        gn = gn_s[idx]
        sb = s_s[idx]
        tw = tw_s[idx]

        pre = (adjf[:, None, :] * (sb[:, :, None] + tw[None, :, :])
               + dn[:, None, :] * ub[None, :, :]
               + gn[:, None, :] * vb[None, :, :]
               + b1b[None, :, :])
        h = jax.nn.sigmoid(pre)

        add_acc = jnp.zeros((_EO * _GC, _NC), jnp.float32)
        subs = []
        for c in range(_NCH):
            hc = h[c * _GC:(c + 1) * _GC].reshape(_GC * _EH, _NC)
            ec = jax.nn.sigmoid(
                jnp.dot(bd, hc, preferred_element_type=jnp.float32) + b2b)
            add_acc = add_acc + ec
            subs.append(jnp.sum(ec.reshape(_EO, _GC, _NC), axis=2))
        add = jnp.sum(add_acc.reshape(_EO, _GC, _NC), axis=1)
        sub = jnp.concatenate(subs, axis=1)
        cmat = add - sub

        o = jax.nn.sigmoid(
            jnp.dot(w3_ref[...], cmat) + b3b_s[...])
        outs.append(o.T[None])
    out_ref[...] = jnp.concatenate(outs, axis=0)


def kernel(x, adj_dist, adj_direct, wind_mean, wind_std, W1, b1, W2, b2, W3,
           b3):
    del wind_mean, wind_std
    f32 = jnp.float32
    bf = jnp.bfloat16

    whole = lambda shape: pl.BlockSpec(shape, lambda b: tuple(
        0 for _ in shape))
    out = pl.pallas_call(
        _fused_kernel,
        grid=(_B // _BB,),
        in_specs=[
            whole((_B, _NC, _NC)),
            whole((_B, _NC, _NC)),
            whole((_B, _NC, _IN)),
            whole((_EH, _EH)),
            whole((1, _EH)),
            whole((1, _EO)),
            whole((1, _OUT)),
            whole((_EO, _EH)),
            whole((_OUT, _EO)),
        ],
        out_specs=pl.BlockSpec((_BB, _NC, _OUT), lambda b: (b, 0, 0)),
        out_shape=jax.ShapeDtypeStruct((_B, _NC, _OUT), f32),
        scratch_shapes=[
            pltpu.VMEM((_B, _NC, _NC), bf),
            pltpu.VMEM((_B, _NC, _NC), bf),
            pltpu.VMEM((_B, _NC, _NC), bf),
            pltpu.VMEM((_B, _NC, _EH), bf),
            pltpu.VMEM((_B, _EH, _NC), bf),
            pltpu.VMEM((3 * _EH, _NC), bf),
            pltpu.VMEM((_EO * _GC, _NC), f32),
            pltpu.VMEM((_OUT, _NC), f32),
            pltpu.VMEM((_EO * _GC, _GC * _EH), bf),
        ],
    )(adj_dist, adj_direct, x, W1, b1.reshape(1, _EH), b2.reshape(1, _EO),
      b3.reshape(1, _OUT), W2, W3)
    return out
